# baseline (speedup 1.0000x reference)
import functools

import jax
import jax.numpy as jnp
from jax.experimental import pallas as pl
from jax.experimental.pallas import tpu as pltpu


def _sae_body(x_ref, wp_ref, we_hbm, be_ref, wd_hbm, bd_ref, out_ref,
              we_v, wd_v, lat_ref, we_sem, wd_sem, *, k, max_iters):
    mi = pl.program_id(0)

    @pl.when(pl.program_id(1) == 0)
    def _load_weights():
        we_cp = pltpu.make_async_copy(we_hbm.at[mi], we_v, we_sem)
        wd_cp = pltpu.make_async_copy(wd_hbm.at[mi], wd_v, wd_sem)
        we_cp.start()
        wd_cp.start()
        we_cp.wait()
        wd_cp.wait()

    x = x_ref[0]
    proj = jnp.dot(x, wp_ref[0], preferred_element_type=jnp.float32)
    z = jnp.dot(proj, we_v[...], preferred_element_type=jnp.float32)
    z = z + be_ref[0, 0, :][None, :]
    lat = jnp.maximum(z, 0.0)
    lat_ref[...] = lat

    h = lat.shape[1]
    kf = jnp.float32(k)
    log_k = jnp.log(kf)
    rmax = jnp.max(lat, axis=1, keepdims=True)

    def count_ge(t):
        return jnp.sum((lat_ref[...] >= t).astype(jnp.float32), axis=1,
                       keepdims=True)

    eps_rel = 2e-6

    def is_active(lo, hi, c_lo):
        return jnp.logical_and(c_lo != kf, hi - lo > eps_rel * hi)

    def cond(carry):
        it, lo, hi, c_lo, c_hi = carry
        return jnp.logical_and(it < max_iters,
                               jnp.any(is_active(lo, hi, c_lo)))

    def body(carry):
        it, lo, hi, c_lo, c_hi = carry
        active = is_active(lo, hi, c_lo)
        llo = jnp.log(jnp.maximum(c_lo, 0.5))
        lhi = jnp.log(jnp.maximum(c_hi, 0.5))
        denom = jnp.maximum(llo - lhi, 1e-6)
        t_int = lo + (hi - lo) * (llo - log_k) / denom
        t_mid = 0.5 * (lo + hi)
        ok_int = jnp.logical_and(t_int > lo, t_int < hi)
        t = jnp.where(ok_int, t_int, t_mid)
        progress = jnp.logical_and(t > lo, t < hi)
        stuck = jnp.logical_and(active, jnp.logical_not(progress))
        act = jnp.logical_and(active, progress)
        c = count_ge(t)
        take = c >= kf
        new_lo = jnp.where(jnp.logical_and(act, take), t, lo)
        new_clo = jnp.where(jnp.logical_and(act, take), c, c_lo)
        new_hi = jnp.where(jnp.logical_and(act, jnp.logical_not(take)), t, hi)
        new_chi = jnp.where(jnp.logical_and(act, jnp.logical_not(take)), c, c_hi)
        new_hi = jnp.where(stuck, new_lo, new_hi)
        return it + 1, new_lo, new_hi, new_clo, new_chi

    init = (jnp.int32(0),
            jnp.zeros_like(rmax),
            rmax,
            jnp.full_like(rmax, float(h)),
            jnp.ones_like(rmax))
    _, lo, _, _, _ = jax.lax.while_loop(cond, body, init)

    masked = jnp.where(lat_ref[...] >= lo, lat_ref[...], 0.0)
    recon = jnp.dot(masked, wd_v[...], preferred_element_type=jnp.float32)
    out_ref[0] = recon + bd_ref[0, 0, :][None, :]


def _fused_sae(x_s, wp_s, we_s, be_s, wd_s, bd_s, *, k, block_rows,
               max_iters=64):
    m, b, in_dim = x_s.shape
    proj_dim = wp_s.shape[2]
    hidden = we_s.shape[2]
    nb = b // block_rows
    grid = (m, nb)
    return pl.pallas_call(
        functools.partial(_sae_body, k=k, max_iters=max_iters),
        grid=grid,
        in_specs=[
            pl.BlockSpec((1, block_rows, in_dim), lambda mi, i: (mi, i, 0)),
            pl.BlockSpec((1, in_dim, proj_dim), lambda mi, i: (mi, 0, 0)),
            pl.BlockSpec(memory_space=pl.ANY),
            pl.BlockSpec((1, 8, hidden), lambda mi, i: (mi, 0, 0)),
            pl.BlockSpec(memory_space=pl.ANY),
            pl.BlockSpec((1, 8, proj_dim), lambda mi, i: (mi, 0, 0)),
        ],
        out_specs=pl.BlockSpec((1, block_rows, proj_dim),
                               lambda mi, i: (mi, i, 0)),
        out_shape=jax.ShapeDtypeStruct((m, b, proj_dim), jnp.float32),
        scratch_shapes=[
            pltpu.VMEM((proj_dim, hidden), jnp.float32),
            pltpu.VMEM((hidden, proj_dim), jnp.float32),
            pltpu.VMEM((block_rows, hidden), jnp.float32),
            pltpu.SemaphoreType.DMA,
            pltpu.SemaphoreType.DMA,
        ],
        compiler_params=pltpu.CompilerParams(
            dimension_semantics=("arbitrary", "arbitrary")),
    )(x_s, wp_s, we_s, be_s, wd_s, bd_s)


def kernel(img, txt, W_img_proj, W_txt_proj, W_img_enc, b_img_enc,
           W_txt_enc, b_txt_enc, W_img_dec, b_img_dec, W_txt_dec, b_txt_dec):
    in_dim = img.shape[1]
    pad = in_dim - txt.shape[1]
    x_s = jnp.stack([img, jnp.pad(txt, ((0, 0), (0, pad)))])
    wp_s = jnp.stack([W_img_proj, jnp.pad(W_txt_proj, ((0, pad), (0, 0)))])
    we_s = jnp.stack([W_img_enc, W_txt_enc])
    be_s = jnp.broadcast_to(
        jnp.stack([b_img_enc, b_txt_enc])[:, None, :], (2, 8, b_img_enc.shape[0]))
    wd_s = jnp.stack([W_img_dec, W_txt_dec])
    bd_s = jnp.broadcast_to(
        jnp.stack([b_img_dec, b_txt_dec])[:, None, :], (2, 8, b_img_dec.shape[0]))
    return _fused_sae(x_s, wp_s, we_s, be_s, wd_s, bd_s, k=64, block_rows=128)


# speedup vs baseline: 1.0046x; 1.0046x over previous
import functools

import jax
import jax.numpy as jnp
from jax.experimental import pallas as pl
from jax.experimental.pallas import tpu as pltpu


def _sae_body(x_ref, wp_ref, we_hbm, be_ref, wd_hbm, bd_ref, out_ref,
              we_v, wd_v, lat_ref, we_sem, wd_sem, *, k, max_iters):
    mi = pl.program_id(0)

    @pl.when(pl.program_id(1) == 0)
    def _load_weights():
        we_cp = pltpu.make_async_copy(we_hbm.at[mi], we_v, we_sem)
        wd_cp = pltpu.make_async_copy(wd_hbm.at[mi], wd_v, wd_sem)
        we_cp.start()
        wd_cp.start()
        we_cp.wait()
        wd_cp.wait()

    x = x_ref[0]
    proj = jnp.dot(x, wp_ref[0], preferred_element_type=jnp.float32)
    z = jnp.dot(proj, we_v[...], preferred_element_type=jnp.float32)
    z = z + be_ref[0, 0, :][None, :]
    lat = jnp.maximum(z, 0.0)
    lat_ref[...] = lat

    h = lat.shape[1]
    kf = jnp.float32(k)
    log_k = jnp.log(kf)
    rmax = jnp.max(lat, axis=1, keepdims=True)

    def count_ge(t):
        return jnp.sum((lat_ref[...] >= t).astype(jnp.float32), axis=1,
                       keepdims=True)

    eps_rel = 2e-6

    def is_active(lo, hi, c_lo):
        return jnp.logical_and(c_lo != kf, hi - lo > eps_rel * hi)

    def cond(carry):
        it, lo, hi, c_lo, c_hi = carry
        return jnp.logical_and(it < max_iters,
                               jnp.any(is_active(lo, hi, c_lo)))

    def body(carry):
        it, lo, hi, c_lo, c_hi = carry
        active = is_active(lo, hi, c_lo)
        llo = jnp.log(jnp.maximum(c_lo, 0.5))
        lhi = jnp.log(jnp.maximum(c_hi, 0.5))
        denom = jnp.maximum(llo - lhi, 1e-6)
        t_int = lo + (hi - lo) * (llo - log_k) / denom
        t_mid = 0.5 * (lo + hi)
        ok_int = jnp.logical_and(t_int > lo, t_int < hi)
        t = jnp.where(ok_int, t_int, t_mid)
        progress = jnp.logical_and(t > lo, t < hi)
        stuck = jnp.logical_and(active, jnp.logical_not(progress))
        act = jnp.logical_and(active, progress)
        c = count_ge(t)
        take = c >= kf
        new_lo = jnp.where(jnp.logical_and(act, take), t, lo)
        new_clo = jnp.where(jnp.logical_and(act, take), c, c_lo)
        new_hi = jnp.where(jnp.logical_and(act, jnp.logical_not(take)), t, hi)
        new_chi = jnp.where(jnp.logical_and(act, jnp.logical_not(take)), c, c_hi)
        new_hi = jnp.where(stuck, new_lo, new_hi)
        return it + 1, new_lo, new_hi, new_clo, new_chi

    init = (jnp.int32(0),
            jnp.zeros_like(rmax),
            rmax,
            jnp.full_like(rmax, float(h)),
            jnp.ones_like(rmax))
    _, lo, _, _, _ = jax.lax.while_loop(cond, body, init)

    masked = jnp.where(lat_ref[...] >= lo, lat_ref[...], 0.0)
    recon = jnp.dot(masked.astype(jnp.bfloat16), wd_v[...],
                    preferred_element_type=jnp.float32)
    out_ref[0] = recon + bd_ref[0, 0, :][None, :]


def _fused_sae(x_s, wp_s, we_s, be_s, wd_s, bd_s, *, k, block_rows,
               max_iters=64):
    m, b, in_dim = x_s.shape
    proj_dim = wp_s.shape[2]
    hidden = we_s.shape[2]
    nb = b // block_rows
    grid = (m, nb)
    return pl.pallas_call(
        functools.partial(_sae_body, k=k, max_iters=max_iters),
        grid=grid,
        in_specs=[
            pl.BlockSpec((1, block_rows, in_dim), lambda mi, i: (mi, i, 0)),
            pl.BlockSpec((1, in_dim, proj_dim), lambda mi, i: (mi, 0, 0)),
            pl.BlockSpec(memory_space=pl.ANY),
            pl.BlockSpec((1, 8, hidden), lambda mi, i: (mi, 0, 0)),
            pl.BlockSpec(memory_space=pl.ANY),
            pl.BlockSpec((1, 8, proj_dim), lambda mi, i: (mi, 0, 0)),
        ],
        out_specs=pl.BlockSpec((1, block_rows, proj_dim),
                               lambda mi, i: (mi, i, 0)),
        out_shape=jax.ShapeDtypeStruct((m, b, proj_dim), jnp.float32),
        scratch_shapes=[
            pltpu.VMEM((proj_dim, hidden), jnp.float32),
            pltpu.VMEM((hidden, proj_dim), jnp.bfloat16),
            pltpu.VMEM((block_rows, hidden), jnp.float32),
            pltpu.SemaphoreType.DMA,
            pltpu.SemaphoreType.DMA,
        ],
        compiler_params=pltpu.CompilerParams(
            dimension_semantics=("arbitrary", "arbitrary")),
    )(x_s, wp_s, we_s, be_s, wd_s, bd_s)


def kernel(img, txt, W_img_proj, W_txt_proj, W_img_enc, b_img_enc,
           W_txt_enc, b_txt_enc, W_img_dec, b_img_dec, W_txt_dec, b_txt_dec):
    in_dim = img.shape[1]
    pad = in_dim - txt.shape[1]
    x_s = jnp.stack([img, jnp.pad(txt, ((0, 0), (0, pad)))])
    wp_s = jnp.stack([W_img_proj, jnp.pad(W_txt_proj, ((0, pad), (0, 0)))])
    we_s = jnp.stack([W_img_enc, W_txt_enc])
    be_s = jnp.broadcast_to(
        jnp.stack([b_img_enc, b_txt_enc])[:, None, :], (2, 8, b_img_enc.shape[0]))
    wd_s = jnp.stack([W_img_dec, W_txt_dec]).astype(jnp.bfloat16)
    bd_s = jnp.broadcast_to(
        jnp.stack([b_img_dec, b_txt_dec])[:, None, :], (2, 8, b_img_dec.shape[0]))
    return _fused_sae(x_s, wp_s, we_s, be_s, wd_s, bd_s, k=64, block_rows=128)
